# baseline (speedup 1.0000x reference)
import functools

import jax
import jax.numpy as jnp
from jax import lax
from jax.experimental import pallas as pl
from jax.experimental.pallas import tpu as pltpu
from jax.experimental.pallas import tpu_sc as plsc

N_NODES_K = 10000
N_EDGES_K = 320000
NC = 2
NS = 16
NW = NC * NS
LANES = 16
EPW = N_EDGES_K // NW
HALF = EPW // 2
HPAD = 10240


def _p1_range(idx_v, cw_v, lo, hi):
  n = hi - lo

  def step(i):
    sl = pl.ds(i * LANES, LANES)
    v = idx_v[sl]
    cnt, last = plsc.scan_count(v)
    cw_v[sl] = jnp.where(last, cnt.astype(jnp.float32), 0.0)

  def body(i, carry):
    step(lo + 3 * i)
    step(lo + 3 * i + 1)
    step(lo + 3 * i + 2)
    return carry

  lax.fori_loop(0, n // 3, body, 0)

  def tail(i, carry):
    step(lo + (n // 3) * 3 + i)
    return carry

  lax.fori_loop(0, n - (n // 3) * 3, tail, 0)


def _hist_body(recv_hbm, zeros_hbm, parts_hbm, idx_v, hist_v, cw_v, sem_i,
               sem_z):
  c = lax.axis_index("c")
  s = lax.axis_index("s")
  wid = s * NC + c
  base = N_EDGES_K + wid * EPW

  idx_dma = pltpu.make_async_copy(recv_hbm.at[pl.ds(base, EPW)], idx_v, sem_i)
  zero_dma = pltpu.make_async_copy(zeros_hbm, hist_v, sem_z)
  idx_dma.start()
  zero_dma.start()

  def p1_step(i):
    sl = pl.ds(i * LANES, LANES)
    v = idx_v[sl]
    cnt, last = plsc.scan_count(v)
    cw_v[sl] = jnp.where(last, cnt.astype(jnp.float32), 0.0)

  def p2_step(i):
    sl = pl.ds(i * LANES, LANES)
    v = idx_v[sl]
    mv = cw_v[sl]
    plsc.addupdate_scatter(hist_v, [v], mv, mask=mv > 0.5)

  nv = EPW // LANES
  U = 5

  idx_dma.wait()
  zero_dma.wait()

  for j in range(U):
    p1_step(j)

  def fused(i, carry):
    for j in range(U):
      p1_step(U * i + U + j)
    for j in range(U):
      p2_step(U * i + j)
    return carry

  lax.fori_loop(0, nv // U - 1, fused, 0)

  for j in range(U):
    p2_step(nv - U + j)

  pltpu.sync_copy(hist_v, parts_hbm.at[wid])


_hist = pl.kernel(
    _hist_body,
    out_type=jax.ShapeDtypeStruct((NW, HPAD), jnp.float32),
    mesh=plsc.VectorSubcoreMesh(
        core_axis_name="c", subcore_axis_name="s", num_cores=NC,
        num_subcores=NS),
    scratch_types=[
        pltpu.VMEM((EPW,), jnp.int32),
        pltpu.VMEM((HPAD,), jnp.float32),
        pltpu.VMEM((EPW,), jnp.float32),
        pltpu.SemaphoreType.DMA,
        pltpu.SemaphoreType.DMA,
    ],
    compiler_params=pltpu.CompilerParams(needs_layout_passes=False),
)


def _merge_body(parts_ref, out_ref):
  out_ref[...] = jnp.sum(parts_ref[...], axis=0, keepdims=True)


_merge = pl.pallas_call(
    _merge_body,
    out_shape=jax.ShapeDtypeStruct((1, HPAD), jnp.float32),
)


@jax.jit
def kernel(x, edge_index):
  n = x.shape[0]
  zeros = jnp.zeros((HPAD,), jnp.float32)
  parts = _hist(edge_index.astype(jnp.int32).reshape(-1), zeros)
  merged = _merge(parts)
  return merged[0, :n].reshape(n, 1)


# speedup vs baseline: 1.2728x; 1.2728x over previous
import functools

import jax
import jax.numpy as jnp
from jax import lax
from jax.experimental import pallas as pl
from jax.experimental.pallas import tpu as pltpu
from jax.experimental.pallas import tpu_sc as plsc

N_NODES_K = 10000
N_EDGES_K = 320000
NC = 2
NS = 16
NW = NC * NS
LANES = 16
EPW = N_EDGES_K // NW
HALF = EPW // 2
HPAD = 10000


def _p1_range(idx_v, cw_v, lo, hi):
  n = hi - lo

  def step(i):
    sl = pl.ds(i * LANES, LANES)
    v = idx_v[sl]
    cnt, last = plsc.scan_count(v)
    cw_v[sl] = jnp.where(last, cnt.astype(jnp.float32), 0.0)

  def body(i, carry):
    step(lo + 3 * i)
    step(lo + 3 * i + 1)
    step(lo + 3 * i + 2)
    return carry

  lax.fori_loop(0, n // 3, body, 0)

  def tail(i, carry):
    step(lo + (n // 3) * 3 + i)
    return carry

  lax.fori_loop(0, n - (n // 3) * 3, tail, 0)


def _hist_body(recv_hbm, zeros_hbm, parts_hbm, idx_v, hist_v, cw_v, sem_i,
               sem_z):
  c = lax.axis_index("c")
  s = lax.axis_index("s")
  wid = s * NC + c
  base = N_EDGES_K + wid * EPW

  idx_dma = pltpu.make_async_copy(recv_hbm.at[pl.ds(base, EPW)], idx_v, sem_i)
  zero_dma = pltpu.make_async_copy(zeros_hbm, hist_v, sem_z)
  idx_dma.start()
  zero_dma.start()

  idx_dma.wait()
  _p1_range(idx_v, cw_v, 0, EPW // LANES)

  zero_dma.wait()

  def p2_step(i):
    sl = pl.ds(i * LANES, LANES)
    v = idx_v[sl]
    mv = cw_v[sl]
    plsc.addupdate_scatter(hist_v, [v], mv, mask=mv > 0.5)

  def pass2(i, carry):
    for j in range(8):
      p2_step(8 * i + j)
    return carry

  lax.fori_loop(0, EPW // (8 * LANES), pass2, 0)

  def p2_tail(i, carry):
    p2_step((EPW // (8 * LANES)) * 8 + i)
    return carry

  lax.fori_loop(0, (EPW // LANES) - (EPW // (8 * LANES)) * 8, p2_tail, 0)

  pltpu.sync_copy(hist_v, parts_hbm.at[wid])


_hist = pl.kernel(
    _hist_body,
    out_type=jax.ShapeDtypeStruct((NW, HPAD), jnp.float32),
    mesh=plsc.VectorSubcoreMesh(
        core_axis_name="c", subcore_axis_name="s", num_cores=NC,
        num_subcores=NS),
    scratch_types=[
        pltpu.VMEM((EPW,), jnp.int32),
        pltpu.VMEM((HPAD,), jnp.float32),
        pltpu.VMEM((EPW,), jnp.float32),
        pltpu.SemaphoreType.DMA,
        pltpu.SemaphoreType.DMA,
    ],
    compiler_params=pltpu.CompilerParams(needs_layout_passes=False),
)


def _merge_body(parts_ref, out_ref):
  out_ref[...] = jnp.sum(parts_ref[...], axis=0, keepdims=True)


_merge = pl.pallas_call(
    _merge_body,
    out_shape=jax.ShapeDtypeStruct((1, HPAD), jnp.float32),
)


@jax.jit
def kernel(x, edge_index):
  n = x.shape[0]
  zeros = jnp.zeros((HPAD,), jnp.float32)
  parts = _hist(edge_index.astype(jnp.int32).reshape(-1), zeros)
  merged = _merge(parts)
  return merged.reshape(n, 1)


# speedup vs baseline: 1.3829x; 1.0865x over previous
import functools

import jax
import jax.numpy as jnp
from jax import lax
from jax.experimental import pallas as pl
from jax.experimental.pallas import tpu as pltpu
from jax.experimental.pallas import tpu_sc as plsc

N_NODES_K = 10000
N_EDGES_K = 320000
NC = 2
NS = 16
NW = NC * NS
LANES = 16
EPW = N_EDGES_K // NW
HALF = EPW // 2
HPAD = 10000


def _p1_range(idx_v, cw_v, lo, hi):
  n = hi - lo

  def step(i):
    sl = pl.ds(i * LANES, LANES)
    v = idx_v[sl]
    cnt, last = plsc.scan_count(v)
    cw_v[sl] = jnp.where(last, cnt.astype(jnp.float32), 0.0)

  def body(i, carry):
    step(lo + 3 * i)
    step(lo + 3 * i + 1)
    step(lo + 3 * i + 2)
    return carry

  lax.fori_loop(0, n // 3, body, 0)

  def tail(i, carry):
    step(lo + (n // 3) * 3 + i)
    return carry

  lax.fori_loop(0, n - (n // 3) * 3, tail, 0)


def _hist_body(recv_hbm, parts_hbm, idx_v, hist_v, cw_v, sem_i):
  c = lax.axis_index("c")
  s = lax.axis_index("s")
  wid = s * NC + c
  base = N_EDGES_K + wid * EPW

  idx_dma = pltpu.make_async_copy(recv_hbm.at[pl.ds(base, EPW)], idx_v, sem_i)
  idx_dma.start()

  zvec = jnp.zeros((LANES,), jnp.float32)

  def zero(i, carry):
    for j in range(8):
      hist_v[pl.ds((8 * i + j) * LANES, LANES)] = zvec
    return carry

  lax.fori_loop(0, HPAD // (8 * LANES), zero, 0)

  def zero_tail(i, carry):
    hist_v[pl.ds(((HPAD // (8 * LANES)) * 8 + i) * LANES, LANES)] = zvec
    return carry

  lax.fori_loop(0, HPAD // LANES - (HPAD // (8 * LANES)) * 8, zero_tail, 0)

  idx_dma.wait()
  _p1_range(idx_v, cw_v, 0, EPW // LANES)

  def p2_step(i):
    sl = pl.ds(i * LANES, LANES)
    v = idx_v[sl]
    mv = cw_v[sl]
    plsc.addupdate_scatter(hist_v, [v], mv, mask=mv > 0.5)

  def pass2(i, carry):
    for j in range(8):
      p2_step(8 * i + j)
    return carry

  lax.fori_loop(0, EPW // (8 * LANES), pass2, 0)

  def p2_tail(i, carry):
    p2_step((EPW // (8 * LANES)) * 8 + i)
    return carry

  lax.fori_loop(0, (EPW // LANES) - (EPW // (8 * LANES)) * 8, p2_tail, 0)

  pltpu.sync_copy(hist_v, parts_hbm.at[wid])


_hist = pl.kernel(
    _hist_body,
    out_type=jax.ShapeDtypeStruct((NW, HPAD), jnp.float32),
    mesh=plsc.VectorSubcoreMesh(
        core_axis_name="c", subcore_axis_name="s", num_cores=NC,
        num_subcores=NS),
    scratch_types=[
        pltpu.VMEM((EPW,), jnp.int32),
        pltpu.VMEM((HPAD,), jnp.float32),
        pltpu.VMEM((EPW,), jnp.float32),
        pltpu.SemaphoreType.DMA,
    ],
    compiler_params=pltpu.CompilerParams(needs_layout_passes=False),
)


def _merge_body(parts_ref, out_ref):
  out_ref[...] = jnp.sum(parts_ref[...], axis=0, keepdims=True)


_merge = pl.pallas_call(
    _merge_body,
    out_shape=jax.ShapeDtypeStruct((1, HPAD), jnp.float32),
)


@jax.jit
def kernel(x, edge_index):
  n = x.shape[0]
  parts = _hist(edge_index.astype(jnp.int32).reshape(-1))
  merged = _merge(parts)
  return merged.reshape(n, 1)


# speedup vs baseline: 1.3917x; 1.0063x over previous
import functools

import jax
import jax.numpy as jnp
from jax import lax
from jax.experimental import pallas as pl
from jax.experimental.pallas import tpu as pltpu
from jax.experimental.pallas import tpu_sc as plsc

N_NODES_K = 10000
N_EDGES_K = 320000
NC = 2
NS = 16
NW = NC * NS
LANES = 16
EPW = N_EDGES_K // NW
HALF = EPW // 2
HPAD = 10000


def _p1_range(idx_v, cw_v, lo, hi):
  n = hi - lo

  def step(i):
    sl = pl.ds(i * LANES, LANES)
    v = idx_v[sl]
    cnt, last = plsc.scan_count(v)
    cw_v[sl] = jnp.where(last, cnt.astype(jnp.float32), 0.0)

  def body(i, carry):
    for j in range(5):
      step(lo + 5 * i + j)
    return carry

  lax.fori_loop(0, n // 5, body, 0)

  def tail(i, carry):
    step(lo + (n // 5) * 5 + i)
    return carry

  lax.fori_loop(0, n - (n // 5) * 5, tail, 0)


def _hist_body(recv_hbm, parts_hbm, idx_v, hist_v, cw_v, sem_i):
  c = lax.axis_index("c")
  s = lax.axis_index("s")
  wid = s * NC + c
  base = N_EDGES_K + wid * EPW

  idx_dma = pltpu.make_async_copy(recv_hbm.at[pl.ds(base, EPW)], idx_v, sem_i)
  idx_dma.start()

  zvec = jnp.zeros((LANES,), jnp.float32)

  def zero(i, carry):
    for j in range(8):
      hist_v[pl.ds((8 * i + j) * LANES, LANES)] = zvec
    return carry

  lax.fori_loop(0, HPAD // (8 * LANES), zero, 0)

  def zero_tail(i, carry):
    hist_v[pl.ds(((HPAD // (8 * LANES)) * 8 + i) * LANES, LANES)] = zvec
    return carry

  lax.fori_loop(0, HPAD // LANES - (HPAD // (8 * LANES)) * 8, zero_tail, 0)

  idx_dma.wait()
  _p1_range(idx_v, cw_v, 0, EPW // LANES)

  def p2_step(i):
    sl = pl.ds(i * LANES, LANES)
    v = idx_v[sl]
    mv = cw_v[sl]
    plsc.addupdate_scatter(hist_v, [v], mv, mask=mv > 0.5)

  def pass2(i, carry):
    for j in range(16):
      p2_step(16 * i + j)
    return carry

  lax.fori_loop(0, EPW // (16 * LANES), pass2, 0)

  def p2_tail(i, carry):
    p2_step((EPW // (16 * LANES)) * 16 + i)
    return carry

  lax.fori_loop(0, (EPW // LANES) - (EPW // (16 * LANES)) * 16, p2_tail, 0)

  pltpu.sync_copy(hist_v, parts_hbm.at[wid])


_hist = pl.kernel(
    _hist_body,
    out_type=jax.ShapeDtypeStruct((NW, HPAD), jnp.float32),
    mesh=plsc.VectorSubcoreMesh(
        core_axis_name="c", subcore_axis_name="s", num_cores=NC,
        num_subcores=NS),
    scratch_types=[
        pltpu.VMEM((EPW,), jnp.int32),
        pltpu.VMEM((HPAD,), jnp.float32),
        pltpu.VMEM((EPW,), jnp.float32),
        pltpu.SemaphoreType.DMA,
    ],
    compiler_params=pltpu.CompilerParams(needs_layout_passes=False),
)


def _merge_body(parts_ref, out_ref):
  out_ref[...] = jnp.sum(parts_ref[...], axis=0, keepdims=True)


_merge = pl.pallas_call(
    _merge_body,
    out_shape=jax.ShapeDtypeStruct((1, HPAD), jnp.float32),
)


@jax.jit
def kernel(x, edge_index):
  n = x.shape[0]
  parts = _hist(edge_index.astype(jnp.int32).reshape(-1))
  merged = _merge(parts)
  return merged.reshape(n, 1)
